# baseline (speedup 1.0000x reference)
import jax
import jax.numpy as jnp
from jax import lax
from jax.experimental import pallas as pl
from jax.experimental.pallas import tpu as pltpu
from jax.experimental.pallas import tpu_sc as plsc

_T = 0.9
_N = 32768
_ROWS = 64
_L = 16
_NSL = _N // _L
_UNROLL = 8
_H1 = 16384
_H3 = 16


def _scan_desc(hist_ref, nslices, acc0, tprime, chunk=1):
    lo_slice = jnp.int32(0)
    top_slice = jnp.int32(nslices)
    found0 = True
    acc = acc0
    if chunk > 1:
        nch = nslices // chunk

        def cond0(c):
            k, _, found = c
            return jnp.logical_and(k < nch, jnp.logical_not(found))

        def body0(c):
            k, acc, _ = c
            base = (nch - 1 - k) * chunk * _L
            s = hist_ref[pl.ds(base, _L)]
            for j in range(1, chunk):
                s = s + hist_ref[pl.ds(base + j * _L, _L)]
            tot = jnp.sum(s)
            hit = acc + tot > tprime
            return (jnp.where(hit, k, k + 1), jnp.where(hit, acc, acc + tot),
                    hit)

        k0, acc, found0 = lax.while_loop(cond0, body0,
                                         (jnp.int32(0), acc0, False))
        win = nch - 1 - jnp.minimum(k0, nch - 1)
        lo_slice = win * chunk
        top_slice = lo_slice + chunk

    def cond(c):
        k, _, found = c
        return jnp.logical_and(k < top_slice, jnp.logical_not(found))

    def body(c):
        k, acc, _ = c
        kk = top_slice - 1 - k
        sl = hist_ref[pl.ds(kk * _L, _L)]
        tot = jnp.sum(sl)
        hit = acc + tot > tprime
        return (jnp.where(hit, k, k + 1), jnp.where(hit, acc, acc + tot), hit)

    if chunk > 1:
        k, acc, found = lax.while_loop(
            cond, body, (jnp.where(found0, 0, top_slice), acc, False))
    else:
        k, acc, found = lax.while_loop(cond, body, (jnp.int32(0), acc, False))

    kk = top_slice - 1 - jnp.minimum(k, top_slice - 1)
    sl = hist_ref[pl.ds(kk * _L, _L)]
    rv = lax.rev(sl, (0,))
    run = acc + plsc.cumsum(rv)
    cross = run > tprime
    lane = jnp.max(plsc.all_reduce_ffs(cross))
    iota16 = lax.broadcasted_iota(jnp.int32, (_L,), 0)
    fexcl = jnp.sum(jnp.where(iota16 == lane, run - rv, 0.0))
    bval = jnp.sum(jnp.where(iota16 == lane, rv, 0.0))
    cut = kk * _L + (_L - 1) - lane
    return found, cut, fexcl, bval


def _zero(ref, n):
    z = jnp.zeros((_L,), jnp.float32)

    def body(i, _):
        base = i * _UNROLL * _L
        for k in range(_UNROLL):
            ref[pl.ds(base + k * _L, _L)] = z
        return 0

    if n // _L >= _UNROLL:
        lax.fori_loop(0, n // (_L * _UNROLL), body, 0)
    else:
        for k in range(n // _L):
            ref[pl.ds(k * _L, _L)] = z


def _process_row(g_v, m_v, hist_v, hist3_v):
    _zero(hist_v, _H1)

    def h1_body(i, acc):
        base = i * _UNROLL * _L
        gs = [g_v[pl.ds(base + k * _L, _L)] for k in range(_UNROLL)]
        ms = [m_v[pl.ds(base + k * _L, _L)] for k in range(_UNROLL)]
        gated = [jnp.where(ms[k] != 0, gs[k], 0.0) for k in range(_UNROLL)]
        pats = [plsc.bitcast(gated[k], jnp.int32) for k in range(_UNROLL)]
        for k in range(_UNROLL):
            g_v[pl.ds(base + k * _L, _L)] = gated[k]
        for k in range(_UNROLL):
            plsc.addupdate_scatter(hist_v,
                                   (lax.shift_right_logical(pats[k], 16),),
                                   gated[k], mask=pats[k] > 0)
        for k in range(_UNROLL):
            acc = acc + gated[k]
        return acc

    acc = lax.fori_loop(0, _NSL // _UNROLL, h1_body,
                        jnp.zeros((_L,), jnp.float32))
    total = jnp.maximum(jnp.sum(acc), jnp.float32(1e-12))
    tprime = jnp.float32(_T) * total

    found1, b1, f1, _ = _scan_desc(hist_v, _H1 // _L, jnp.float32(0.0), tprime,
                                   chunk=8)
    allkeep = jnp.logical_not(found1)

    _zero(hist_v, _H1)

    def h2_body(i, _):
        base = i * _UNROLL * _L
        gated = [g_v[pl.ds(base + k * _L, _L)] for k in range(_UNROLL)]
        pats = [plsc.bitcast(gated[k], jnp.int32) for k in range(_UNROLL)]
        sels = [lax.shift_right_logical(pats[k], 16) == b1
                for k in range(_UNROLL)]
        idxs = [jnp.bitwise_and(lax.shift_right_logical(pats[k], 2), _H1 - 1)
                for k in range(_UNROLL)]
        for k in range(_UNROLL):
            plsc.addupdate_scatter(hist_v, (idxs[k],), gated[k], mask=sels[k])
        return 0

    lax.fori_loop(0, _NSL // _UNROLL, h2_body, 0)
    _, b2, f2, bsum2 = _scan_desc(hist_v, _H1 // _L, f1, tprime, chunk=8)
    pref = jnp.bitwise_or(lax.shift_left(b1, 14), b2)

    v_apx = plsc.bitcast(
        jnp.full((_L,), jnp.bitwise_or(lax.shift_left(pref, 2), 2),
                 jnp.int32), jnp.float32)
    e24 = (jnp.full((_L,), bsum2) / jnp.maximum(v_apx, jnp.float32(1e-30))
           + 0.5).astype(jnp.int32)
    multi = jnp.max(e24) > 1

    def resolve_multi(_):
        _zero(hist3_v, _H3)

        def h3_body(i, _):
            base = i * _UNROLL * _L
            gated = [g_v[pl.ds(base + k * _L, _L)] for k in range(_UNROLL)]
            pats = [plsc.bitcast(gated[k], jnp.int32) for k in range(_UNROLL)]
            sels = [lax.shift_right_logical(pats[k], 2) == pref
                    for k in range(_UNROLL)]
            idxs = [jnp.bitwise_and(pats[k], 3) for k in range(_UNROLL)]
            for k in range(_UNROLL):
                plsc.addupdate_scatter(hist3_v, (idxs[k],), gated[k],
                                       mask=sels[k])
            return 0

        lax.fori_loop(0, _NSL // _UNROLL, h3_body, 0)
        _, b3, f_hi3, bsum3 = _scan_desc(hist3_v, _H3 // _L, f2, tprime)
        vpat = jnp.bitwise_or(lax.shift_left(pref, 2), b3)
        vstar = plsc.bitcast(jnp.full((_L,), vpat, jnp.int32), jnp.float32)
        e3 = (jnp.full((_L,), bsum3) / jnp.maximum(vstar, jnp.float32(1e-30))
              + 0.5).astype(jnp.int32)
        return vpat, vpat - 1, f_hi3, vstar, e3

    def resolve_single(_):
        gt_thresh = jnp.bitwise_or(lax.shift_left(pref, 2), 3)
        eq_lo = lax.shift_left(pref, 2) - 1
        vstar = jnp.full((_L,), bsum2)
        return gt_thresh, eq_lo, f2, vstar, jnp.full((_L,), 1, jnp.int32)

    gt_thresh, eq_lo, f_hi, vstar, ev = lax.cond(
        multi, resolve_multi, resolve_single, 0)
    gt_tv = jnp.full((_L,), gt_thresh)
    eq_lov = jnp.full((_L,), eq_lo)

    rfv = jnp.full((_L,), tprime - f_hi) / jnp.maximum(vstar, jnp.float32(1e-30))
    rv = jnp.clip(rfv, 0.0, jnp.float32(_N)).astype(jnp.int32)
    rv = jnp.where(jnp.full((_L,), f_hi) == 0.0, jnp.maximum(rv, 1), rv)
    akm = jnp.full((_L,), jnp.where(allkeep, 1, 0)) != 0
    need_rank = jnp.logical_and(jnp.max(rv) > 0, jnp.max(rv) < jnp.max(ev))

    def out_fast(_):
        eq_keep = rv >= ev

        def body(i, _):
            base = i * _UNROLL * _L
            gated = [g_v[pl.ds(base + k * _L, _L)] for k in range(_UNROLL)]
            ms = [m_v[pl.ds(base + k * _L, _L)] for k in range(_UNROLL)]
            outs = []
            for k in range(_UNROLL):
                pat = plsc.bitcast(gated[k], jnp.int32)
                gt = pat > gt_tv
                eq = (pat > eq_lov) & jnp.logical_not(gt)
                keep = (gt | (eq & eq_keep) | akm) & (ms[k] != 0)
                outs.append(jnp.where(keep, 1.0, 0.0))
            for k in range(_UNROLL):
                g_v[pl.ds(base + k * _L, _L)] = outs[k]
            return 0

        lax.fori_loop(0, _NSL // _UNROLL, body, 0)
        return 0

    def out_slow(_):
        def body(i, carry):
            base = i * _UNROLL * _L
            gated = [g_v[pl.ds(base + k * _L, _L)] for k in range(_UNROLL)]
            ms = [m_v[pl.ds(base + k * _L, _L)] for k in range(_UNROLL)]
            pats = [plsc.bitcast(gated[k], jnp.int32) for k in range(_UNROLL)]
            gts = [pats[k] > gt_tv for k in range(_UNROLL)]
            eqs = [(pats[k] > eq_lov) & jnp.logical_not(gts[k])
                   for k in range(_UNROLL)]
            css = [plsc.cumsum(jnp.where(eqs[k], 1, 0)) for k in range(_UNROLL)]
            pcs = [plsc.all_reduce_population_count(eqs[k])
                   for k in range(_UNROLL)]
            outs = []
            for k in range(_UNROLL):
                rank = css[k] + carry
                keep = (gts[k] | (eqs[k] & (rank <= rv)) | akm) & (ms[k] != 0)
                outs.append(jnp.where(keep, 1.0, 0.0))
                carry = carry + pcs[k]
            for k in range(_UNROLL):
                g_v[pl.ds(base + k * _L, _L)] = outs[k]
            return carry

        lax.fori_loop(0, _NSL // _UNROLL, body, jnp.zeros((_L,), jnp.int32))
        return 0

    lax.cond(need_rank, out_slow, out_fast, 0)


def _sc_body(g_hbm, m_hbm, o_hbm, ga_v, gb_v, m_v, hist_v, hist3_v,
             sem_g, sem_o):
    wid = lax.axis_index("s") * 2 + lax.axis_index("c")
    row_a = wid * 2
    row_b = row_a + 1

    cp_b = pltpu.async_copy(g_hbm.at[row_b], gb_v, sem_g)
    pltpu.sync_copy(g_hbm.at[row_a], ga_v)
    pltpu.sync_copy(m_hbm.at[row_a], m_v)
    _process_row(ga_v, m_v, hist_v, hist3_v)
    cp_oa = pltpu.async_copy(ga_v, o_hbm.at[row_a], sem_o)
    pltpu.sync_copy(m_hbm.at[row_b], m_v)
    cp_b.wait()
    _process_row(gb_v, m_v, hist_v, hist3_v)
    pltpu.sync_copy(gb_v, o_hbm.at[row_b])
    cp_oa.wait()


def kernel(gates, attention_mask):
    mesh = plsc.VectorSubcoreMesh(core_axis_name="c", subcore_axis_name="s")
    f = pl.kernel(
        _sc_body,
        out_type=jax.ShapeDtypeStruct((_ROWS, _N), jnp.float32),
        mesh=mesh,
        compiler_params=pltpu.CompilerParams(needs_layout_passes=False),
        scratch_types=[
            pltpu.VMEM((_N,), jnp.float32),
            pltpu.VMEM((_N,), jnp.float32),
            pltpu.VMEM((_N,), jnp.int32),
            pltpu.VMEM((_H1,), jnp.float32),
            pltpu.VMEM((_H3,), jnp.float32),
            pltpu.SemaphoreType.DMA,
            pltpu.SemaphoreType.DMA,
        ],
    )
    out = f(gates, attention_mask)
    return out.astype(jnp.bool_)


# speedup vs baseline: 1.0636x; 1.0636x over previous
import jax
import jax.numpy as jnp
from jax import lax
from jax.experimental import pallas as pl
from jax.experimental.pallas import tpu as pltpu
from jax.experimental.pallas import tpu_sc as plsc

_T = 0.9
_N = 32768
_ROWS = 64
_L = 16
_NSL = _N // _L
_UNROLL = 8
_H1 = 4096
_H3 = 64


def _scan_desc(hist_ref, nslices, acc0, tprime, chunk=1):
    lo_slice = jnp.int32(0)
    top_slice = jnp.int32(nslices)
    found0 = True
    acc = acc0
    if chunk > 1:
        nch = nslices // chunk

        def cond0(c):
            k, _, found = c
            return jnp.logical_and(k < nch, jnp.logical_not(found))

        def body0(c):
            k, acc, _ = c
            base = (nch - 1 - k) * chunk * _L
            s = hist_ref[pl.ds(base, _L)]
            for j in range(1, chunk):
                s = s + hist_ref[pl.ds(base + j * _L, _L)]
            tot = jnp.sum(s)
            hit = acc + tot > tprime
            return (jnp.where(hit, k, k + 1), jnp.where(hit, acc, acc + tot),
                    hit)

        k0, acc, found0 = lax.while_loop(cond0, body0,
                                         (jnp.int32(0), acc0, False))
        win = nch - 1 - jnp.minimum(k0, nch - 1)
        lo_slice = win * chunk
        top_slice = lo_slice + chunk

    def cond(c):
        k, _, found = c
        return jnp.logical_and(k < top_slice, jnp.logical_not(found))

    def body(c):
        k, acc, _ = c
        kk = top_slice - 1 - k
        sl = hist_ref[pl.ds(kk * _L, _L)]
        tot = jnp.sum(sl)
        hit = acc + tot > tprime
        return (jnp.where(hit, k, k + 1), jnp.where(hit, acc, acc + tot), hit)

    if chunk > 1:
        k, acc, found = lax.while_loop(
            cond, body, (jnp.where(found0, 0, top_slice), acc, False))
    else:
        k, acc, found = lax.while_loop(cond, body, (jnp.int32(0), acc, False))

    kk = top_slice - 1 - jnp.minimum(k, top_slice - 1)
    sl = hist_ref[pl.ds(kk * _L, _L)]
    rv = lax.rev(sl, (0,))
    run = acc + plsc.cumsum(rv)
    cross = run > tprime
    lane = jnp.max(plsc.all_reduce_ffs(cross))
    iota16 = lax.broadcasted_iota(jnp.int32, (_L,), 0)
    fexcl = jnp.sum(jnp.where(iota16 == lane, run - rv, 0.0))
    bval = jnp.sum(jnp.where(iota16 == lane, rv, 0.0))
    cut = kk * _L + (_L - 1) - lane
    return found, cut, fexcl, bval


def _zero(ref, n):
    z = jnp.zeros((_L,), jnp.float32)

    def body(i, _):
        base = i * _UNROLL * _L
        for k in range(_UNROLL):
            ref[pl.ds(base + k * _L, _L)] = z
        return 0

    if n // _L >= _UNROLL:
        lax.fori_loop(0, n // (_L * _UNROLL), body, 0)
    else:
        for k in range(n // _L):
            ref[pl.ds(k * _L, _L)] = z


def _process_row(g_v, m_v, hist_v, hist3_v):
    _zero(hist_v, _H1)

    def h1_body(i, acc):
        base = i * _UNROLL * _L
        gs = [g_v[pl.ds(base + k * _L, _L)] for k in range(_UNROLL)]
        ms = [m_v[pl.ds(base + k * _L, _L)] for k in range(_UNROLL)]
        gated = [jnp.where(ms[k] != 0, gs[k], 0.0) for k in range(_UNROLL)]
        pats = [plsc.bitcast(gated[k], jnp.int32) for k in range(_UNROLL)]
        for k in range(_UNROLL):
            g_v[pl.ds(base + k * _L, _L)] = gated[k]
        for k in range(_UNROLL):
            plsc.addupdate_scatter(hist_v,
                                   (lax.shift_right_logical(pats[k], 18),),
                                   gated[k], mask=pats[k] > 0)
        for k in range(_UNROLL):
            acc = acc + gated[k]
        return acc

    acc = lax.fori_loop(0, _NSL // _UNROLL, h1_body,
                        jnp.zeros((_L,), jnp.float32))
    total = jnp.maximum(jnp.sum(acc), jnp.float32(1e-12))
    tprime = jnp.float32(_T) * total

    found1, b1, f1, _ = _scan_desc(hist_v, _H1 // _L, jnp.float32(0.0), tprime,
                                   chunk=8)
    allkeep = jnp.logical_not(found1)

    _zero(hist_v, _H1)

    def h2_body(i, _):
        base = i * _UNROLL * _L
        gated = [g_v[pl.ds(base + k * _L, _L)] for k in range(_UNROLL)]
        pats = [plsc.bitcast(gated[k], jnp.int32) for k in range(_UNROLL)]
        sels = [lax.shift_right_logical(pats[k], 18) == b1
                for k in range(_UNROLL)]
        idxs = [jnp.bitwise_and(lax.shift_right_logical(pats[k], 6), _H1 - 1)
                for k in range(_UNROLL)]
        for k in range(_UNROLL):
            plsc.addupdate_scatter(hist_v, (idxs[k],), gated[k], mask=sels[k])
        return 0

    lax.fori_loop(0, _NSL // _UNROLL, h2_body, 0)
    _, b2, f2, bsum2 = _scan_desc(hist_v, _H1 // _L, f1, tprime, chunk=8)
    pref = jnp.bitwise_or(lax.shift_left(b1, 12), b2)

    v_apx = plsc.bitcast(
        jnp.full((_L,), jnp.bitwise_or(lax.shift_left(pref, 6), 32),
                 jnp.int32), jnp.float32)
    e24 = (jnp.full((_L,), bsum2) / jnp.maximum(v_apx, jnp.float32(1e-30))
           + 0.5).astype(jnp.int32)
    multi = jnp.max(e24) > 1

    def resolve_multi(_):
        _zero(hist3_v, _H3)

        def h3_body(i, _):
            base = i * _UNROLL * _L
            gated = [g_v[pl.ds(base + k * _L, _L)] for k in range(_UNROLL)]
            pats = [plsc.bitcast(gated[k], jnp.int32) for k in range(_UNROLL)]
            sels = [lax.shift_right_logical(pats[k], 6) == pref
                    for k in range(_UNROLL)]
            idxs = [jnp.bitwise_and(pats[k], _H3 - 1) for k in range(_UNROLL)]
            for k in range(_UNROLL):
                plsc.addupdate_scatter(hist3_v, (idxs[k],), gated[k],
                                       mask=sels[k])
            return 0

        lax.fori_loop(0, _NSL // _UNROLL, h3_body, 0)
        _, b3, f_hi3, bsum3 = _scan_desc(hist3_v, _H3 // _L, f2, tprime)
        vpat = jnp.bitwise_or(lax.shift_left(pref, 6), b3)
        vstar = plsc.bitcast(jnp.full((_L,), vpat, jnp.int32), jnp.float32)
        e3 = (jnp.full((_L,), bsum3) / jnp.maximum(vstar, jnp.float32(1e-30))
              + 0.5).astype(jnp.int32)
        return vpat, vpat - 1, f_hi3, vstar, e3

    def resolve_single(_):
        gt_thresh = jnp.bitwise_or(lax.shift_left(pref, 6), _H3 - 1)
        eq_lo = lax.shift_left(pref, 6) - 1
        vstar = jnp.full((_L,), bsum2)
        return gt_thresh, eq_lo, f2, vstar, jnp.full((_L,), 1, jnp.int32)

    gt_thresh, eq_lo, f_hi, vstar, ev = lax.cond(
        multi, resolve_multi, resolve_single, 0)
    gt_tv = jnp.full((_L,), gt_thresh)
    eq_lov = jnp.full((_L,), eq_lo)

    rfv = jnp.full((_L,), tprime - f_hi) / jnp.maximum(vstar, jnp.float32(1e-30))
    rv = jnp.clip(rfv, 0.0, jnp.float32(_N)).astype(jnp.int32)
    rv = jnp.where(jnp.full((_L,), f_hi) == 0.0, jnp.maximum(rv, 1), rv)
    akm = jnp.full((_L,), jnp.where(allkeep, 1, 0)) != 0
    need_rank = jnp.logical_and(jnp.max(rv) > 0, jnp.max(rv) < jnp.max(ev))

    def out_fast(_):
        eq_keep = rv >= ev

        def body(i, _):
            base = i * _UNROLL * _L
            gated = [g_v[pl.ds(base + k * _L, _L)] for k in range(_UNROLL)]
            ms = [m_v[pl.ds(base + k * _L, _L)] for k in range(_UNROLL)]
            outs = []
            for k in range(_UNROLL):
                pat = plsc.bitcast(gated[k], jnp.int32)
                gt = pat > gt_tv
                eq = (pat > eq_lov) & jnp.logical_not(gt)
                keep = (gt | (eq & eq_keep) | akm) & (ms[k] != 0)
                outs.append(jnp.where(keep, 1.0, 0.0))
            for k in range(_UNROLL):
                g_v[pl.ds(base + k * _L, _L)] = outs[k]
            return 0

        lax.fori_loop(0, _NSL // _UNROLL, body, 0)
        return 0

    def out_slow(_):
        def body(i, carry):
            base = i * _UNROLL * _L
            gated = [g_v[pl.ds(base + k * _L, _L)] for k in range(_UNROLL)]
            ms = [m_v[pl.ds(base + k * _L, _L)] for k in range(_UNROLL)]
            pats = [plsc.bitcast(gated[k], jnp.int32) for k in range(_UNROLL)]
            gts = [pats[k] > gt_tv for k in range(_UNROLL)]
            eqs = [(pats[k] > eq_lov) & jnp.logical_not(gts[k])
                   for k in range(_UNROLL)]
            css = [plsc.cumsum(jnp.where(eqs[k], 1, 0)) for k in range(_UNROLL)]
            pcs = [plsc.all_reduce_population_count(eqs[k])
                   for k in range(_UNROLL)]
            outs = []
            for k in range(_UNROLL):
                rank = css[k] + carry
                keep = (gts[k] | (eqs[k] & (rank <= rv)) | akm) & (ms[k] != 0)
                outs.append(jnp.where(keep, 1.0, 0.0))
                carry = carry + pcs[k]
            for k in range(_UNROLL):
                g_v[pl.ds(base + k * _L, _L)] = outs[k]
            return carry

        lax.fori_loop(0, _NSL // _UNROLL, body, jnp.zeros((_L,), jnp.int32))
        return 0

    lax.cond(need_rank, out_slow, out_fast, 0)


def _sc_body(g_hbm, m_hbm, o_hbm, ga_v, gb_v, m_v, hist_v, hist3_v,
             sem_g, sem_o):
    wid = lax.axis_index("s") * 2 + lax.axis_index("c")
    row_a = wid * 2
    row_b = row_a + 1

    cp_b = pltpu.async_copy(g_hbm.at[row_b], gb_v, sem_g)
    pltpu.sync_copy(g_hbm.at[row_a], ga_v)
    pltpu.sync_copy(m_hbm.at[row_a], m_v)
    _process_row(ga_v, m_v, hist_v, hist3_v)
    cp_oa = pltpu.async_copy(ga_v, o_hbm.at[row_a], sem_o)
    pltpu.sync_copy(m_hbm.at[row_b], m_v)
    cp_b.wait()
    _process_row(gb_v, m_v, hist_v, hist3_v)
    pltpu.sync_copy(gb_v, o_hbm.at[row_b])
    cp_oa.wait()


def kernel(gates, attention_mask):
    mesh = plsc.VectorSubcoreMesh(core_axis_name="c", subcore_axis_name="s")
    f = pl.kernel(
        _sc_body,
        out_type=jax.ShapeDtypeStruct((_ROWS, _N), jnp.float32),
        mesh=mesh,
        compiler_params=pltpu.CompilerParams(needs_layout_passes=False),
        scratch_types=[
            pltpu.VMEM((_N,), jnp.float32),
            pltpu.VMEM((_N,), jnp.float32),
            pltpu.VMEM((_N,), jnp.int32),
            pltpu.VMEM((_H1,), jnp.float32),
            pltpu.VMEM((_H3,), jnp.float32),
            pltpu.SemaphoreType.DMA,
            pltpu.SemaphoreType.DMA,
        ],
    )
    out = f(gates, attention_mask)
    return out.astype(jnp.bool_)


# speedup vs baseline: 1.1244x; 1.0571x over previous
import jax
import jax.numpy as jnp
from jax import lax
from jax.experimental import pallas as pl
from jax.experimental.pallas import tpu as pltpu
from jax.experimental.pallas import tpu_sc as plsc

_T = 0.9
_N = 32768
_ROWS = 64
_L = 16
_NSL = _N // _L
_UNROLL = 8
_H1 = 4096
_H3 = 64


def _scan_desc(hist_ref, nslices, acc0, tprime, chunk=1):
    lo_slice = jnp.int32(0)
    top_slice = jnp.int32(nslices)
    found0 = True
    acc = acc0
    if chunk > 1:
        nch = nslices // chunk

        def cond0(c):
            k, _, found = c
            return jnp.logical_and(k < nch, jnp.logical_not(found))

        def body0(c):
            k, acc, _ = c
            base = (nch - 1 - k) * chunk * _L
            s = hist_ref[pl.ds(base, _L)]
            for j in range(1, chunk):
                s = s + hist_ref[pl.ds(base + j * _L, _L)]
            tot = jnp.sum(s)
            hit = acc + tot > tprime
            return (jnp.where(hit, k, k + 1), jnp.where(hit, acc, acc + tot),
                    hit)

        k0, acc, found0 = lax.while_loop(cond0, body0,
                                         (jnp.int32(0), acc0, False))
        win = nch - 1 - jnp.minimum(k0, nch - 1)
        lo_slice = win * chunk
        top_slice = lo_slice + chunk

    def cond(c):
        k, _, found = c
        return jnp.logical_and(k < top_slice, jnp.logical_not(found))

    def body(c):
        k, acc, _ = c
        kk = top_slice - 1 - k
        sl = hist_ref[pl.ds(kk * _L, _L)]
        tot = jnp.sum(sl)
        hit = acc + tot > tprime
        return (jnp.where(hit, k, k + 1), jnp.where(hit, acc, acc + tot), hit)

    if chunk > 1:
        k, acc, found = lax.while_loop(
            cond, body, (jnp.where(found0, 0, top_slice), acc, False))
    else:
        k, acc, found = lax.while_loop(cond, body, (jnp.int32(0), acc, False))

    kk = top_slice - 1 - jnp.minimum(k, top_slice - 1)
    sl = hist_ref[pl.ds(kk * _L, _L)]
    rv = lax.rev(sl, (0,))
    run = acc + plsc.cumsum(rv)
    cross = run > tprime
    lane = jnp.max(plsc.all_reduce_ffs(cross))
    iota16 = lax.broadcasted_iota(jnp.int32, (_L,), 0)
    fexcl = jnp.sum(jnp.where(iota16 == lane, run - rv, 0.0))
    bval = jnp.sum(jnp.where(iota16 == lane, rv, 0.0))
    cut = kk * _L + (_L - 1) - lane
    return found, cut, fexcl, bval


def _zero(ref, n):
    z = jnp.zeros((_L,), jnp.float32)

    def body(i, _):
        base = i * _UNROLL * _L
        for k in range(_UNROLL):
            ref[pl.ds(base + k * _L, _L)] = z
        return 0

    if n // _L >= _UNROLL:
        lax.fori_loop(0, n // (_L * _UNROLL), body, 0)
    else:
        for k in range(n // _L):
            ref[pl.ds(k * _L, _L)] = z


def _process_row(g_v, m_v, hist_v, hist3_v):
    _zero(hist_v, _H1)

    def h1_body(i, acc):
        base = i * _UNROLL * _L
        gs = [g_v[pl.ds(base + k * _L, _L)] for k in range(_UNROLL)]
        ms = [m_v[pl.ds(base + k * _L, _L)] for k in range(_UNROLL)]
        gated = [jnp.where(ms[k] != 0, gs[k], 0.0) for k in range(_UNROLL)]
        pats = [plsc.bitcast(gated[k], jnp.int32) for k in range(_UNROLL)]
        for k in range(_UNROLL):
            g_v[pl.ds(base + k * _L, _L)] = gated[k]
        for k in range(_UNROLL):
            plsc.addupdate_scatter(hist_v,
                                   (lax.shift_right_logical(pats[k], 18),),
                                   gated[k], mask=pats[k] > 0)
        for k in range(_UNROLL):
            acc = acc + gated[k]
        return acc

    acc = lax.fori_loop(0, _NSL // _UNROLL, h1_body,
                        jnp.zeros((_L,), jnp.float32))
    total = jnp.maximum(jnp.sum(acc), jnp.float32(1e-12))
    tprime = jnp.float32(_T) * total

    found1, b1, f1, _ = _scan_desc(hist_v, _H1 // _L, jnp.float32(0.0), tprime,
                                   chunk=8)
    allkeep = jnp.logical_not(found1)

    _zero(hist_v, _H1)

    def h2_body(i, _):
        base = i * _UNROLL * _L
        gated = [g_v[pl.ds(base + k * _L, _L)] for k in range(_UNROLL)]
        pats = [plsc.bitcast(gated[k], jnp.int32) for k in range(_UNROLL)]
        sels = [lax.shift_right_logical(pats[k], 18) == b1
                for k in range(_UNROLL)]
        idxs = [jnp.bitwise_and(lax.shift_right_logical(pats[k], 6), _H1 - 1)
                for k in range(_UNROLL)]
        for k in range(_UNROLL):
            plsc.addupdate_scatter(hist_v, (idxs[k],), gated[k], mask=sels[k])
        return 0

    lax.fori_loop(0, _NSL // _UNROLL, h2_body, 0)
    _, b2, f2, bsum2 = _scan_desc(hist_v, _H1 // _L, f1, tprime, chunk=8)
    pref = jnp.bitwise_or(lax.shift_left(b1, 12), b2)

    v_apx = plsc.bitcast(
        jnp.full((_L,), jnp.bitwise_or(lax.shift_left(pref, 6), 32),
                 jnp.int32), jnp.float32)
    e24 = (jnp.full((_L,), bsum2) / jnp.maximum(v_apx, jnp.float32(1e-30))
           + 0.5).astype(jnp.int32)
    multi = jnp.max(e24) > 1

    def resolve_multi(_):
        _zero(hist3_v, _H3)

        def h3_body(i, _):
            base = i * _UNROLL * _L
            gated = [g_v[pl.ds(base + k * _L, _L)] for k in range(_UNROLL)]
            pats = [plsc.bitcast(gated[k], jnp.int32) for k in range(_UNROLL)]
            sels = [lax.shift_right_logical(pats[k], 6) == pref
                    for k in range(_UNROLL)]
            idxs = [jnp.bitwise_and(pats[k], _H3 - 1) for k in range(_UNROLL)]
            for k in range(_UNROLL):
                plsc.addupdate_scatter(hist3_v, (idxs[k],), gated[k],
                                       mask=sels[k])
            return 0

        lax.fori_loop(0, _NSL // _UNROLL, h3_body, 0)
        _, b3, f_hi3, bsum3 = _scan_desc(hist3_v, _H3 // _L, f2, tprime)
        vpat = jnp.bitwise_or(lax.shift_left(pref, 6), b3)
        vstar = plsc.bitcast(jnp.full((_L,), vpat, jnp.int32), jnp.float32)
        e3 = (jnp.full((_L,), bsum3) / jnp.maximum(vstar, jnp.float32(1e-30))
              + 0.5).astype(jnp.int32)
        return vpat, vpat - 1, f_hi3, vstar, e3

    def resolve_single(_):
        gt_thresh = jnp.bitwise_or(lax.shift_left(pref, 6), _H3 - 1)
        eq_lo = lax.shift_left(pref, 6) - 1
        vstar = jnp.full((_L,), bsum2)
        return gt_thresh, eq_lo, f2, vstar, jnp.full((_L,), 1, jnp.int32)

    gt_thresh, eq_lo, f_hi, vstar, ev = lax.cond(
        multi, resolve_multi, resolve_single, 0)
    gt_tv = jnp.full((_L,), gt_thresh)
    eq_lov = jnp.full((_L,), eq_lo)

    rfv = jnp.full((_L,), tprime - f_hi) / jnp.maximum(vstar, jnp.float32(1e-30))
    rv = jnp.clip(rfv, 0.0, jnp.float32(_N)).astype(jnp.int32)
    rv = jnp.where(jnp.full((_L,), f_hi) == 0.0, jnp.maximum(rv, 1), rv)
    need_rank = jnp.logical_and(jnp.max(rv) > 0, jnp.max(rv) < jnp.max(ev))

    def out_allkeep(_):
        def body(i, _):
            base = i * _UNROLL * _L
            ms = [m_v[pl.ds(base + k * _L, _L)] for k in range(_UNROLL)]
            outs = [jnp.where(ms[k] != 0, 1.0, 0.0) for k in range(_UNROLL)]
            for k in range(_UNROLL):
                g_v[pl.ds(base + k * _L, _L)] = outs[k]
            return 0

        lax.fori_loop(0, _NSL // _UNROLL, body, 0)
        return 0

    def out_fast(_):
        eq_keep = rv >= ev

        def body(i, _):
            base = i * _UNROLL * _L
            gated = [g_v[pl.ds(base + k * _L, _L)] for k in range(_UNROLL)]
            outs = []
            for k in range(_UNROLL):
                pat = plsc.bitcast(gated[k], jnp.int32)
                gt = pat > gt_tv
                eq = (pat > eq_lov) & jnp.logical_not(gt)
                keep = gt | (eq & eq_keep)
                outs.append(jnp.where(keep, 1.0, 0.0))
            for k in range(_UNROLL):
                g_v[pl.ds(base + k * _L, _L)] = outs[k]
            return 0

        lax.fori_loop(0, _NSL // _UNROLL, body, 0)
        return 0

    def out_slow(_):
        def body(i, carry):
            base = i * _UNROLL * _L
            gated = [g_v[pl.ds(base + k * _L, _L)] for k in range(_UNROLL)]
            pats = [plsc.bitcast(gated[k], jnp.int32) for k in range(_UNROLL)]
            gts = [pats[k] > gt_tv for k in range(_UNROLL)]
            eqs = [(pats[k] > eq_lov) & jnp.logical_not(gts[k])
                   for k in range(_UNROLL)]
            css = [plsc.cumsum(jnp.where(eqs[k], 1, 0)) for k in range(_UNROLL)]
            pcs = [plsc.all_reduce_population_count(eqs[k])
                   for k in range(_UNROLL)]
            outs = []
            for k in range(_UNROLL):
                rank = css[k] + carry
                keep = gts[k] | (eqs[k] & (rank <= rv))
                outs.append(jnp.where(keep, 1.0, 0.0))
                carry = carry + pcs[k]
            for k in range(_UNROLL):
                g_v[pl.ds(base + k * _L, _L)] = outs[k]
            return carry

        lax.fori_loop(0, _NSL // _UNROLL, body, jnp.zeros((_L,), jnp.int32))
        return 0

    def out_select(_):
        lax.cond(need_rank, out_slow, out_fast, 0)
        return 0

    lax.cond(allkeep, out_allkeep, out_select, 0)


def _sc_body(g_hbm, m_hbm, o_hbm, ga_v, gb_v, m_v, hist_v, hist3_v,
             sem_g, sem_o):
    wid = lax.axis_index("s") * 2 + lax.axis_index("c")
    row_a = wid * 2
    row_b = row_a + 1

    cp_b = pltpu.async_copy(g_hbm.at[row_b], gb_v, sem_g)
    pltpu.sync_copy(g_hbm.at[row_a], ga_v)
    pltpu.sync_copy(m_hbm.at[row_a], m_v)
    _process_row(ga_v, m_v, hist_v, hist3_v)
    cp_oa = pltpu.async_copy(ga_v, o_hbm.at[row_a], sem_o)
    pltpu.sync_copy(m_hbm.at[row_b], m_v)
    cp_b.wait()
    _process_row(gb_v, m_v, hist_v, hist3_v)
    pltpu.sync_copy(gb_v, o_hbm.at[row_b])
    cp_oa.wait()


def kernel(gates, attention_mask):
    mesh = plsc.VectorSubcoreMesh(core_axis_name="c", subcore_axis_name="s")
    f = pl.kernel(
        _sc_body,
        out_type=jax.ShapeDtypeStruct((_ROWS, _N), jnp.float32),
        mesh=mesh,
        compiler_params=pltpu.CompilerParams(needs_layout_passes=False),
        scratch_types=[
            pltpu.VMEM((_N,), jnp.float32),
            pltpu.VMEM((_N,), jnp.float32),
            pltpu.VMEM((_N,), jnp.int32),
            pltpu.VMEM((_H1,), jnp.float32),
            pltpu.VMEM((_H3,), jnp.float32),
            pltpu.SemaphoreType.DMA,
            pltpu.SemaphoreType.DMA,
        ],
    )
    out = f(gates, attention_mask)
    return out.astype(jnp.bool_)
